# Initial kernel scaffold; baseline (speedup 1.0000x reference)
import functools

import jax
import jax.numpy as jnp
from jax import lax
from jax.experimental import pallas as pl
from jax.experimental.pallas import tpu as pltpu
from jax.experimental.pallas import tpu_sc as plsc

NSEG = 10000
NNZ = 320000
D = 128
CHUNK = 128
NCHUNKS = NNZ // CHUNK
NW = 32
BASE_K = NCHUNKS // NW
EXTRA = NCHUNKS - NW * BASE_K
TILE_ROWS = NSEG // 16
ZCHUNK = 125


def _sc_partial_sums(values, ids):
    mesh = plsc.VectorSubcoreMesh(core_axis_name="c", subcore_axis_name="s")

    @functools.partial(
        pl.kernel,
        out_type=jax.ShapeDtypeStruct((2 * NSEG, D), jnp.float32),
        mesh=mesh,
        scratch_types=[
            pltpu.VMEM_SHARED((NSEG, D), jnp.float32),
            pltpu.VMEM((CHUNK, D), jnp.float32),
            pltpu.VMEM((CHUNK,), jnp.int32),
            pltpu.VMEM((ZCHUNK, D), jnp.float32),
        ],
    )
    def k(values_hbm, ids_hbm, out_hbm, acc, vbuf, ibuf, zbuf):
        c = lax.axis_index("c")
        s = lax.axis_index("s")
        wid = s * 2 + c

        @pl.loop(0, ZCHUNK)
        def _zero(i):
            for j in range(D // 16):
                zbuf[i, pl.ds(j * 16, 16)] = jnp.zeros((16,), jnp.float32)

        for i in range(TILE_ROWS // ZCHUNK):
            pltpu.sync_copy(
                zbuf, acc.at[pl.ds(s * TILE_ROWS + i * ZCHUNK, ZCHUNK)])
        plsc.subcore_barrier()

        nk = BASE_K + jnp.where(wid < EXTRA, 1, 0)

        @pl.loop(0, nk)
        def _body(kk):
            base = (wid + NW * kk) * CHUNK
            pltpu.sync_copy(values_hbm.at[pl.ds(base, CHUNK)], vbuf)
            pltpu.sync_copy(ids_hbm.at[pl.ds(base, CHUNK)], ibuf)
            pltpu.sync_copy(vbuf, acc.at[ibuf], add=True)

        plsc.subcore_barrier()

        for i in range(TILE_ROWS // ZCHUNK):
            off = s * TILE_ROWS + i * ZCHUNK
            pltpu.sync_copy(acc.at[pl.ds(off, ZCHUNK)], zbuf)
            pltpu.sync_copy(zbuf, out_hbm.at[pl.ds(c * NSEG + off, ZCHUNK)])

    return k(values, ids)


def _combine(parts):
    def body(p_ref, o_ref):
        o_ref[...] = p_ref[0] + p_ref[1]

    nblk = 8
    return pl.pallas_call(
        body,
        grid=(nblk,),
        in_specs=[pl.BlockSpec((2, NSEG // nblk, D), lambda i: (0, i, 0))],
        out_specs=pl.BlockSpec((NSEG // nblk, D), lambda i: (i, 0)),
        out_shape=jax.ShapeDtypeStruct((NSEG, D), jnp.float32),
    )(parts)


def kernel(values, segment_ids):
    ids = segment_ids.astype(jnp.int32)
    parts = _sc_partial_sums(values, ids)
    return _combine(parts.reshape(2, NSEG, D))


# speedup vs baseline: 4.5353x; 4.5353x over previous
import functools

import jax
import jax.numpy as jnp
from jax import lax
from jax.experimental import pallas as pl
from jax.experimental.pallas import tpu as pltpu
from jax.experimental.pallas import tpu_sc as plsc

NSEG = 10000
NNZ = 320000
D = 128
CHUNK = 128
NCHUNKS = NNZ // CHUNK
NW = 32
BASE_K = NCHUNKS // NW
EXTRA = NCHUNKS - NW * BASE_K
TILE_Z = 624
REM_OFF = 16 * TILE_Z
REM = NSEG - REM_OFF
ZSIZES = (128, 128, 128, 128, 112)


def _sc_partial_sums(values, ids):
    mesh = plsc.VectorSubcoreMesh(core_axis_name="c", subcore_axis_name="s")

    @functools.partial(
        pl.kernel,
        out_type=jax.ShapeDtypeStruct((2 * NSEG, D), jnp.float32),
        mesh=mesh,
        scratch_types=[
            pltpu.VMEM_SHARED((NSEG, D), jnp.float32),
            pltpu.VMEM((CHUNK, D), jnp.float32),
            pltpu.VMEM((CHUNK,), jnp.int32),
            pltpu.VMEM((128, D), jnp.float32),
        ],
    )
    def k(values_hbm, ids_hbm, out_hbm, acc, vbuf, ibuf, zbuf):
        c = lax.axis_index("c")
        s = lax.axis_index("s")
        wid = s * 2 + c

        @pl.loop(0, 128)
        def _zero(i):
            for j in range(D // 16):
                zbuf[i, pl.ds(j * 16, 16)] = jnp.zeros((16,), jnp.float32)

        off = 0
        for sz in ZSIZES:
            pltpu.sync_copy(
                zbuf.at[pl.ds(0, sz)], acc.at[pl.ds(s * TILE_Z + off, sz)])
            off += sz

        @pl.when(s == 0)
        def _zero_rem():
            pltpu.sync_copy(zbuf.at[pl.ds(0, REM)], acc.at[pl.ds(REM_OFF, REM)])

        plsc.subcore_barrier()

        nk = BASE_K + jnp.where(wid < EXTRA, 1, 0)

        @pl.loop(0, nk)
        def _body(kk):
            base = (wid + NW * kk) * CHUNK
            pltpu.sync_copy(values_hbm.at[pl.ds(base, CHUNK)], vbuf)
            pltpu.sync_copy(ids_hbm.at[pl.ds(base, CHUNK)], ibuf)
            pltpu.sync_copy(vbuf, acc.at[ibuf], add=True)

        plsc.subcore_barrier()

        off = 0
        for sz in ZSIZES:
            row = s * TILE_Z + off
            pltpu.sync_copy(acc.at[pl.ds(row, sz)], zbuf.at[pl.ds(0, sz)])
            pltpu.sync_copy(
                zbuf.at[pl.ds(0, sz)], out_hbm.at[pl.ds(c * NSEG + row, sz)])
            off += sz

        @pl.when(s == 0)
        def _wb_rem():
            pltpu.sync_copy(acc.at[pl.ds(REM_OFF, REM)], zbuf.at[pl.ds(0, REM)])
            pltpu.sync_copy(
                zbuf.at[pl.ds(0, REM)],
                out_hbm.at[pl.ds(c * NSEG + REM_OFF, REM)])

    return k(values, ids)


def _combine(parts):
    def body(p_ref, o_ref):
        o_ref[...] = p_ref[0] + p_ref[1]

    nblk = 10
    return pl.pallas_call(
        body,
        grid=(nblk,),
        in_specs=[pl.BlockSpec((2, NSEG // nblk, D), lambda i: (0, i, 0))],
        out_specs=pl.BlockSpec((NSEG // nblk, D), lambda i: (i, 0)),
        out_shape=jax.ShapeDtypeStruct((NSEG, D), jnp.float32),
    )(parts)


def kernel(values, segment_ids):
    ids = segment_ids.astype(jnp.int32)
    parts = _sc_partial_sums(values, ids)
    return _combine(parts.reshape(2, NSEG, D))


# speedup vs baseline: 7.7560x; 1.7101x over previous
import functools

import jax
import jax.numpy as jnp
from jax import lax
from jax.experimental import pallas as pl
from jax.experimental.pallas import tpu as pltpu
from jax.experimental.pallas import tpu_sc as plsc

NSEG = 10000
NNZ = 320000
D = 128
CHUNK = 128
NCHUNKS = NNZ // CHUNK
NW = 32
BASE_K = NCHUNKS // NW
EXTRA = NCHUNKS - NW * BASE_K
TILE_Z = 624
REM_OFF = 16 * TILE_Z
REM = NSEG - REM_OFF
ZSIZES = (128, 128, 128, 128, 112)


def _sc_partial_sums(values, ids):
    mesh = plsc.VectorSubcoreMesh(core_axis_name="c", subcore_axis_name="s")

    @functools.partial(
        pl.kernel,
        out_type=jax.ShapeDtypeStruct((2 * NSEG, D), jnp.float32),
        mesh=mesh,
        scratch_types=[
            pltpu.VMEM_SHARED((NSEG, D), jnp.float32),
            pltpu.VMEM((CHUNK, D), jnp.float32),
            pltpu.VMEM((CHUNK, D), jnp.float32),
            pltpu.VMEM((CHUNK,), jnp.int32),
            pltpu.VMEM((CHUNK,), jnp.int32),
            pltpu.VMEM((128, D), jnp.float32),
            pltpu.SemaphoreType.DMA,
            pltpu.SemaphoreType.DMA,
        ],
    )
    def k(values_hbm, ids_hbm, out_hbm, acc, vbuf0, vbuf1, ibuf0, ibuf1,
          zbuf, sem0, sem1):
        c = lax.axis_index("c")
        s = lax.axis_index("s")
        wid = s * 2 + c

        @pl.loop(0, 128)
        def _zero(i):
            for j in range(D // 16):
                zbuf[i, pl.ds(j * 16, 16)] = jnp.zeros((16,), jnp.float32)

        off = 0
        for sz in ZSIZES:
            pltpu.sync_copy(
                zbuf.at[pl.ds(0, sz)], acc.at[pl.ds(s * TILE_Z + off, sz)])
            off += sz

        @pl.when(s == 0)
        def _zero_rem():
            pltpu.sync_copy(zbuf.at[pl.ds(0, REM)], acc.at[pl.ds(REM_OFF, REM)])

        plsc.subcore_barrier()

        nk = BASE_K + jnp.where(wid < EXTRA, 1, 0)
        bufs = ((vbuf0, ibuf0, sem0), (vbuf1, ibuf1, sem1))

        def _load(kk, vb, ib, sem):
            @pl.when(kk < nk)
            def _():
                base = (wid + NW * kk) * CHUNK
                pltpu.async_copy(values_hbm.at[pl.ds(base, CHUNK)], vb, sem)
                pltpu.async_copy(ids_hbm.at[pl.ds(base, CHUNK)], ib, sem)

        for b in range(2):
            _load(b, *bufs[b])

        @pl.loop(0, BASE_K + 2, step=2)
        def _body(kk0):
            for b in range(2):
                vb, ib, sem = bufs[b]
                kk = kk0 + b

                @pl.when(kk < nk)
                def _():
                    pltpu.make_async_copy(
                        values_hbm.at[pl.ds(0, CHUNK)], vb, sem).wait()
                    pltpu.make_async_copy(
                        ids_hbm.at[pl.ds(0, CHUNK)], ib, sem).wait()
                    pltpu.sync_copy(vb, acc.at[ib], add=True)
                    _load(kk + 2, vb, ib, sem)

        plsc.subcore_barrier()

        off = 0
        for sz in ZSIZES:
            row = s * TILE_Z + off
            pltpu.sync_copy(acc.at[pl.ds(row, sz)], zbuf.at[pl.ds(0, sz)])
            pltpu.sync_copy(
                zbuf.at[pl.ds(0, sz)], out_hbm.at[pl.ds(c * NSEG + row, sz)])
            off += sz

        @pl.when(s == 0)
        def _wb_rem():
            pltpu.sync_copy(acc.at[pl.ds(REM_OFF, REM)], zbuf.at[pl.ds(0, REM)])
            pltpu.sync_copy(
                zbuf.at[pl.ds(0, REM)],
                out_hbm.at[pl.ds(c * NSEG + REM_OFF, REM)])

    return k(values, ids)


def _combine(parts):
    def body(p_ref, o_ref):
        o_ref[...] = p_ref[0] + p_ref[1]

    nblk = 10
    return pl.pallas_call(
        body,
        grid=(nblk,),
        in_specs=[pl.BlockSpec((2, NSEG // nblk, D), lambda i: (0, i, 0))],
        out_specs=pl.BlockSpec((NSEG // nblk, D), lambda i: (i, 0)),
        out_shape=jax.ShapeDtypeStruct((NSEG, D), jnp.float32),
    )(parts)


def kernel(values, segment_ids):
    ids = segment_ids.astype(jnp.int32)
    parts = _sc_partial_sums(values, ids)
    return _combine(parts.reshape(2, NSEG, D))


# speedup vs baseline: 8.0248x; 1.0347x over previous
import functools

import jax
import jax.numpy as jnp
from jax import lax
from jax.experimental import pallas as pl
from jax.experimental.pallas import tpu as pltpu
from jax.experimental.pallas import tpu_sc as plsc

NSEG = 10000
NNZ = 320000
D = 128
HALF = NSEG // 2
CHUNK = 128
NCHUNKS = NNZ // CHUNK
NS = 16
CNT_CHUNK = 4000
CNT_PER_TILE = NNZ // NS
TILE_W = 312
WREM_OFF = NS * TILE_W
WREM = HALF - WREM_OFF
WSIZES = (64, 64, 64, 64, 56)


def _sc_segment_sum(values, ids):
    mesh = plsc.VectorSubcoreMesh(core_axis_name="c", subcore_axis_name="s")

    @functools.partial(
        pl.kernel,
        out_type=jax.ShapeDtypeStruct((NSEG, D), jnp.float32),
        mesh=mesh,
        scratch_types=[
            pltpu.VMEM_SHARED((NSEG, D), jnp.float32),
            pltpu.VMEM((CHUNK, D), jnp.float32),
            pltpu.VMEM((CHUNK, D), jnp.float32),
            pltpu.VMEM((CHUNK,), jnp.int32),
            pltpu.VMEM((CHUNK,), jnp.int32),
            pltpu.VMEM((64, D), jnp.float32),
            pltpu.VMEM((CNT_CHUNK,), jnp.int32),
            pltpu.SMEM((1,), jnp.int32),
            pltpu.SemaphoreType.DMA,
            pltpu.SemaphoreType.DMA,
        ],
    )
    def k(values_hbm, ids_hbm, out_hbm, acc, vbuf0, vbuf1, ibuf0, ibuf1,
          zbuf, cbuf, cnt_smem, sem0, sem1):
        c = lax.axis_index("c")
        s = lax.axis_index("s")

        @pl.when(s == 0)
        def _zero_cnt():
            cnt_smem[0] = 0

        plsc.subcore_barrier()

        ibuf0[pl.ds(0, 16)] = jnp.zeros((16,), jnp.int32)

        @pl.loop(0, CNT_PER_TILE // CNT_CHUNK)
        def _cnt(p):
            pltpu.sync_copy(
                ids_hbm.at[pl.ds(s * CNT_PER_TILE + p * CNT_CHUNK, CNT_CHUNK)],
                cbuf)

            @pl.loop(0, CNT_CHUNK // 16)
            def _inner(i):
                iv = cbuf[pl.ds(i * 16, 16)]
                one = jnp.ones((16,), jnp.int32)
                zero = jnp.zeros((16,), jnp.int32)
                ibuf0[pl.ds(0, 16)] = (
                    ibuf0[pl.ds(0, 16)] + jnp.where(iv < HALF, one, zero))

        cv = ibuf0[pl.ds(0, 16)]
        my_cnt = cv[0]
        for l in range(1, 16):
            my_cnt = my_cnt + cv[l]
        plsc.fetch_and_add(cnt_smem, my_cnt, subcore_id=0)
        plsc.subcore_barrier()
        lb = plsc.fetch_and_add(cnt_smem, 0, subcore_id=0)

        cs = jnp.where(c == 0, 0, lb // CHUNK)
        ce = jnp.where(c == 0, (lb + CHUNK - 1) // CHUNK, NCHUNKS)
        nk = jnp.maximum(ce - cs - s, 0)
        nk = (nk + NS - 1) // NS
        bufs = ((vbuf0, ibuf0, sem0), (vbuf1, ibuf1, sem1))

        def _load(kk, vb, ib, sem):
            @pl.when(kk < nk)
            def _():
                base = (cs + s + NS * kk) * CHUNK
                pltpu.async_copy(values_hbm.at[pl.ds(base, CHUNK)], vb, sem)
                pltpu.async_copy(ids_hbm.at[pl.ds(base, CHUNK)], ib, sem)

        for b in range(2):
            _load(b, *bufs[b])

        @pl.loop(0, 64)
        def _zero(i):
            for j in range(D // 16):
                zbuf[i, pl.ds(j * 16, 16)] = jnp.zeros((16,), jnp.float32)

        off = 0
        for sz in WSIZES:
            pltpu.sync_copy(
                zbuf.at[pl.ds(0, sz)],
                acc.at[pl.ds(c * HALF + s * TILE_W + off, sz)])
            off += sz

        @pl.when(s == NS - 1)
        def _zero_rem():
            pltpu.sync_copy(
                zbuf.at[pl.ds(0, WREM)],
                acc.at[pl.ds(c * HALF + WREM_OFF, WREM)])

        plsc.subcore_barrier()

        nk2 = ((nk + 1) // 2) * 2

        @pl.loop(0, nk2, step=2)
        def _body(kk0):
            for b in range(2):
                vb, ib, sem = bufs[b]
                kk = kk0 + b

                @pl.when(kk < nk)
                def _():
                    pltpu.make_async_copy(
                        values_hbm.at[pl.ds(0, CHUNK)], vb, sem).wait()
                    pltpu.make_async_copy(
                        ids_hbm.at[pl.ds(0, CHUNK)], ib, sem).wait()
                    pltpu.sync_copy(vb, acc.at[ib], add=True)
                    _load(kk + 2, vb, ib, sem)

        plsc.subcore_barrier()

        off = 0
        for sz in WSIZES:
            row = c * HALF + s * TILE_W + off
            pltpu.sync_copy(acc.at[pl.ds(row, sz)], zbuf.at[pl.ds(0, sz)])
            pltpu.sync_copy(zbuf.at[pl.ds(0, sz)], out_hbm.at[pl.ds(row, sz)])
            off += sz

        @pl.when(s == NS - 1)
        def _wb_rem():
            row = c * HALF + WREM_OFF
            pltpu.sync_copy(acc.at[pl.ds(row, WREM)], zbuf.at[pl.ds(0, WREM)])
            pltpu.sync_copy(zbuf.at[pl.ds(0, WREM)], out_hbm.at[pl.ds(row, WREM)])

    return k(values, ids)


def kernel(values, segment_ids):
    ids = segment_ids.astype(jnp.int32)
    return _sc_segment_sum(values, ids)


# speedup vs baseline: 8.9844x; 1.1196x over previous
import functools

import jax
import jax.numpy as jnp
from jax import lax
from jax.experimental import pallas as pl
from jax.experimental.pallas import tpu as pltpu
from jax.experimental.pallas import tpu_sc as plsc

NSEG = 10000
NNZ = 320000
D = 128
HALF = NSEG // 2
CHUNK = 128
NCHUNKS = NNZ // CHUNK
NS = 16
NSLOT = 4
LOOKAHEAD = 3
FPAD = 2560
FPT = FPAD // NS
TRASH = HALF
ACC_ROWS = HALF + 8
TILE_W = 312
WREM_OFF = NS * TILE_W
WREM = HALF - WREM_OFF
WSIZES = (64, 64, 64, 64, 56)


def _sc_segment_sum(values, ids, firsts):
    mesh = plsc.VectorSubcoreMesh(core_axis_name="c", subcore_axis_name="s")

    @functools.partial(
        pl.kernel,
        out_type=jax.ShapeDtypeStruct((NSEG, D), jnp.float32),
        mesh=mesh,
        scratch_types=[
            pltpu.VMEM_SHARED((ACC_ROWS, D), jnp.float32),
            *[pltpu.VMEM((CHUNK, D), jnp.float32) for _ in range(NSLOT)],
            *[pltpu.VMEM((CHUNK,), jnp.int32) for _ in range(NSLOT)],
            pltpu.VMEM((64, D), jnp.float32),
            pltpu.VMEM((FPT,), jnp.int32),
            pltpu.SMEM((1,), jnp.int32),
            *[pltpu.SemaphoreType.DMA for _ in range(NSLOT)],
            *[pltpu.SemaphoreType.DMA for _ in range(NSLOT)],
        ],
    )
    def k(values_hbm, ids_hbm, firsts_hbm, out_hbm, acc,
          vb0, vb1, vb2, vb3, ib0, ib1, ib2, ib3, zbuf, cbuf, cnt_smem,
          ls0, ls1, ls2, ls3, ss0, ss1, ss2, ss3):
        c = lax.axis_index("c")
        s = lax.axis_index("s")
        vbufs = (vb0, vb1, vb2, vb3)
        ibufs = (ib0, ib1, ib2, ib3)
        lsems = (ls0, ls1, ls2, ls3)
        ssems = (ss0, ss1, ss2, ss3)

        @pl.when(s == 0)
        def _zero_cnt():
            cnt_smem[0] = 0

        plsc.subcore_barrier()
        pltpu.sync_copy(firsts_hbm.at[pl.ds(s * FPT, FPT)], cbuf)
        ib0[pl.ds(0, 16)] = jnp.zeros((16,), jnp.int32)

        @pl.loop(0, FPT // 16)
        def _cnt(i):
            iv = cbuf[pl.ds(i * 16, 16)]
            one = jnp.ones((16,), jnp.int32)
            zero = jnp.zeros((16,), jnp.int32)
            ib0[pl.ds(0, 16)] = (
                ib0[pl.ds(0, 16)] + jnp.where(iv < HALF, one, zero))

        cv = ib0[pl.ds(0, 16)]
        my_cnt = cv[0]
        for l in range(1, 16):
            my_cnt = my_cnt + cv[l]
        plsc.fetch_and_add(cnt_smem, my_cnt, subcore_id=0)
        plsc.subcore_barrier()
        c0 = plsc.fetch_and_add(cnt_smem, 0, subcore_id=0)

        cs = jnp.where(c == 0, 0, jnp.maximum(c0 - 1, 0))
        ce = jnp.where(c == 0, c0, NCHUNKS)
        nk = jnp.maximum(ce - cs - s, 0)
        nk = (nk + NS - 1) // NS
        lo = c * HALF

        def _load(kk, slot):
            @pl.when(kk < nk)
            def _():
                base = (cs + s + NS * kk) * CHUNK
                pltpu.async_copy(
                    values_hbm.at[pl.ds(base, CHUNK)], vbufs[slot],
                    lsems[slot])
                pltpu.async_copy(
                    ids_hbm.at[pl.ds(base, CHUNK)], ibufs[slot], lsems[slot])

        for kk in range(LOOKAHEAD):
            _load(kk, kk)

        @pl.loop(0, 64)
        def _zero(i):
            for j in range(D // 16):
                zbuf[i, pl.ds(j * 16, 16)] = jnp.zeros((16,), jnp.float32)

        off = 0
        for sz in WSIZES:
            pltpu.sync_copy(
                zbuf.at[pl.ds(0, sz)], acc.at[pl.ds(s * TILE_W + off, sz)])
            off += sz

        @pl.when(s == NS - 1)
        def _zero_rem():
            pltpu.sync_copy(
                zbuf.at[pl.ds(0, WREM)], acc.at[pl.ds(WREM_OFF, WREM)])

        plsc.subcore_barrier()

        nk4 = ((nk + NSLOT - 1) // NSLOT) * NSLOT

        @pl.loop(0, nk4, step=NSLOT)
        def _body(kk0):
            for b in range(NSLOT):
                vb, ib = vbufs[b], ibufs[b]
                kk = kk0 + b
                nxt = (b + LOOKAHEAD) % NSLOT

                @pl.when(kk < nk)
                def _():
                    pltpu.make_async_copy(
                        values_hbm.at[pl.ds(0, CHUNK)], vb, lsems[b]).wait()
                    pltpu.make_async_copy(
                        ids_hbm.at[pl.ds(0, CHUNK)], ib, lsems[b]).wait()
                    for j in range(CHUNK // 16):
                        iv = ib[pl.ds(j * 16, 16)]
                        loc = iv - lo
                        ok = (loc >= 0) & (loc < HALF)
                        trash = jnp.full((16,), TRASH, jnp.int32)
                        ib[pl.ds(j * 16, 16)] = jnp.where(ok, loc, trash)
                    pltpu.sync_copy(vb, acc.at[ib], add=True)

                    _load(kk + LOOKAHEAD, nxt)

        plsc.subcore_barrier()

        off = 0
        for sz in WSIZES:
            row = s * TILE_W + off
            pltpu.sync_copy(acc.at[pl.ds(row, sz)], zbuf.at[pl.ds(0, sz)])
            pltpu.sync_copy(
                zbuf.at[pl.ds(0, sz)], out_hbm.at[pl.ds(lo + row, sz)])
            off += sz

        @pl.when(s == NS - 1)
        def _wb_rem():
            pltpu.sync_copy(
                acc.at[pl.ds(WREM_OFF, WREM)], zbuf.at[pl.ds(0, WREM)])
            pltpu.sync_copy(
                zbuf.at[pl.ds(0, WREM)],
                out_hbm.at[pl.ds(lo + WREM_OFF, WREM)])

    return k(values, ids, firsts)


def kernel(values, segment_ids):
    ids = segment_ids.astype(jnp.int32)
    firsts = jnp.pad(
        ids.reshape(NCHUNKS, CHUNK)[:, 0], (0, FPAD - NCHUNKS),
        constant_values=NSEG)
    return _sc_segment_sum(values, ids, firsts)


# speedup vs baseline: 9.0836x; 1.0110x over previous
import functools

import jax
import jax.numpy as jnp
from jax import lax
from jax.experimental import pallas as pl
from jax.experimental.pallas import tpu as pltpu
from jax.experimental.pallas import tpu_sc as plsc

NSEG = 10000
NNZ = 320000
D = 128
HALF = NSEG // 2
CHUNK = 128
NCHUNKS = NNZ // CHUNK
NS = 16
NSLOT = 4
LOOKAHEAD = 3
FPAD = 2560
FPT = FPAD // NS
TRASH = HALF
ACC_ROWS = HALF + 8
TILE_W = 312
WREM_OFF = NS * TILE_W
WREM = HALF - WREM_OFF
WSIZES = (64, 64, 64, 64, 56)


def _sc_segment_sum(values, ids, firsts):
    mesh = plsc.VectorSubcoreMesh(core_axis_name="c", subcore_axis_name="s")

    @functools.partial(
        pl.kernel,
        out_type=jax.ShapeDtypeStruct((NSEG, D), jnp.float32),
        mesh=mesh,
        scratch_types=[
            pltpu.VMEM_SHARED((ACC_ROWS, D), jnp.float32),
            *[pltpu.VMEM((CHUNK, D), jnp.float32) for _ in range(NSLOT)],
            *[pltpu.VMEM((CHUNK,), jnp.int32) for _ in range(NSLOT)],
            pltpu.VMEM((64, D), jnp.float32),
            pltpu.VMEM((FPT,), jnp.int32),
            pltpu.SMEM((1,), jnp.int32),
            *[pltpu.SemaphoreType.DMA for _ in range(NSLOT)],
            *[pltpu.SemaphoreType.DMA for _ in range(NSLOT)],
        ],
    )
    def k(values_hbm, ids_hbm, firsts_hbm, out_hbm, acc,
          vb0, vb1, vb2, vb3, ib0, ib1, ib2, ib3, zbuf, cbuf, cnt_smem,
          ls0, ls1, ls2, ls3, ss0, ss1, ss2, ss3):
        c = lax.axis_index("c")
        s = lax.axis_index("s")
        vbufs = (vb0, vb1, vb2, vb3)
        ibufs = (ib0, ib1, ib2, ib3)
        lsems = (ls0, ls1, ls2, ls3)
        ssems = (ss0, ss1, ss2, ss3)

        @pl.when(s == 0)
        def _zero_cnt():
            cnt_smem[0] = 0

        plsc.subcore_barrier()
        pltpu.sync_copy(firsts_hbm.at[pl.ds(s * FPT, FPT)], cbuf)
        ib0[pl.ds(0, 16)] = jnp.zeros((16,), jnp.int32)

        @pl.loop(0, FPT // 16)
        def _cnt(i):
            iv = cbuf[pl.ds(i * 16, 16)]
            one = jnp.ones((16,), jnp.int32)
            zero = jnp.zeros((16,), jnp.int32)
            ib0[pl.ds(0, 16)] = (
                ib0[pl.ds(0, 16)] + jnp.where(iv < HALF, one, zero))

        cv = ib0[pl.ds(0, 16)]
        my_cnt = cv[0]
        for l in range(1, 16):
            my_cnt = my_cnt + cv[l]
        plsc.fetch_and_add(cnt_smem, my_cnt, subcore_id=0)
        plsc.subcore_barrier()
        c0 = plsc.fetch_and_add(cnt_smem, 0, subcore_id=0)

        cs = jnp.where(c == 0, 0, jnp.maximum(c0 - 1, 0))
        ce = jnp.where(c == 0, c0, NCHUNKS)
        nk = jnp.maximum(ce - cs - s, 0)
        nk = (nk + NS - 1) // NS
        lo = c * HALF

        def _load(kk, slot):
            @pl.when(kk < nk)
            def _():
                base = (cs + s + NS * kk) * CHUNK
                pltpu.async_copy(
                    values_hbm.at[pl.ds(base, CHUNK)], vbufs[slot],
                    lsems[slot])
                pltpu.async_copy(
                    ids_hbm.at[pl.ds(base, CHUNK)], ibufs[slot], lsems[slot])

        for kk in range(LOOKAHEAD):
            _load(kk, kk)

        @pl.loop(0, 64)
        def _zero(i):
            for j in range(D // 16):
                zbuf[i, pl.ds(j * 16, 16)] = jnp.zeros((16,), jnp.float32)

        off = 0
        for sz in WSIZES:
            pltpu.sync_copy(
                zbuf.at[pl.ds(0, sz)], acc.at[pl.ds(s * TILE_W + off, sz)])
            off += sz

        @pl.when(s == NS - 1)
        def _zero_rem():
            pltpu.sync_copy(
                zbuf.at[pl.ds(0, WREM)], acc.at[pl.ds(WREM_OFF, WREM)])

        plsc.subcore_barrier()

        nk4 = ((nk + NSLOT - 1) // NSLOT) * NSLOT

        @pl.loop(0, nk4, step=NSLOT)
        def _body(kk0):
            for b in range(NSLOT):
                vb, ib = vbufs[b], ibufs[b]
                kk = kk0 + b
                nxt = (b + LOOKAHEAD) % NSLOT

                @pl.when(kk < nk)
                def _():
                    pltpu.make_async_copy(
                        values_hbm.at[pl.ds(0, CHUNK)], vb, lsems[b]).wait()
                    pltpu.make_async_copy(
                        ids_hbm.at[pl.ds(0, CHUNK)], ib, lsems[b]).wait()
                    for j in range(CHUNK // 16):
                        iv = ib[pl.ds(j * 16, 16)]
                        loc = iv - lo
                        ok = (loc >= 0) & (loc < HALF)
                        trash = jnp.full((16,), TRASH, jnp.int32)
                        ib[pl.ds(j * 16, 16)] = jnp.where(ok, loc, trash)
                    pltpu.async_copy(vb, acc.at[ib], ssems[b], add=True)

                    @pl.when(kk >= 1)
                    def _():
                        pltpu.make_async_copy(
                            vbufs[nxt], acc.at[ibufs[nxt]], ssems[nxt]).wait()

                    _load(kk + LOOKAHEAD, nxt)

        @pl.when(nk > 0)
        def _drain():
            last = (nk - 1) % NSLOT
            for b in range(NSLOT):
                @pl.when(last == b)
                def _():
                    pltpu.make_async_copy(
                        vbufs[b], acc.at[ibufs[b]], ssems[b]).wait()

        plsc.subcore_barrier()

        off = 0
        for sz in WSIZES:
            row = s * TILE_W + off
            pltpu.sync_copy(acc.at[pl.ds(row, sz)], zbuf.at[pl.ds(0, sz)])
            pltpu.sync_copy(
                zbuf.at[pl.ds(0, sz)], out_hbm.at[pl.ds(lo + row, sz)])
            off += sz

        @pl.when(s == NS - 1)
        def _wb_rem():
            pltpu.sync_copy(
                acc.at[pl.ds(WREM_OFF, WREM)], zbuf.at[pl.ds(0, WREM)])
            pltpu.sync_copy(
                zbuf.at[pl.ds(0, WREM)],
                out_hbm.at[pl.ds(lo + WREM_OFF, WREM)])

    return k(values, ids, firsts)


def kernel(values, segment_ids):
    ids = segment_ids.astype(jnp.int32)
    firsts = jnp.pad(
        ids.reshape(NCHUNKS, CHUNK)[:, 0], (0, FPAD - NCHUNKS),
        constant_values=NSEG)
    return _sc_segment_sum(values, ids, firsts)


# speedup vs baseline: 9.1097x; 1.0029x over previous
import functools

import jax
import jax.numpy as jnp
from jax import lax
from jax.experimental import pallas as pl
from jax.experimental.pallas import tpu as pltpu
from jax.experimental.pallas import tpu_sc as plsc

NSEG = 10000
NNZ = 320000
D = 128
HALF = NSEG // 2
CHUNK = 128
NCHUNKS = NNZ // CHUNK
NS = 16
NSLOT = 4
LOOKAHEAD = 3
FPAD = 2560
FPT = FPAD // NS
TRASH = HALF
ACC_ROWS = HALF + 8
TILE_W = 312
WREM_OFF = NS * TILE_W
WREM = HALF - WREM_OFF
WSIZES = (64, 64, 64, 64, 56)


def _sc_segment_sum(values, ids, firsts):
    mesh = plsc.VectorSubcoreMesh(core_axis_name="c", subcore_axis_name="s")

    @functools.partial(
        pl.kernel,
        out_type=jax.ShapeDtypeStruct((NSEG, D), jnp.float32),
        mesh=mesh,
        scratch_types=[
            pltpu.VMEM_SHARED((ACC_ROWS, D), jnp.float32),
            *[pltpu.VMEM((CHUNK, D), jnp.float32) for _ in range(NSLOT)],
            *[pltpu.VMEM((CHUNK,), jnp.int32) for _ in range(NSLOT)],
            pltpu.VMEM((64, D), jnp.float32),
            pltpu.VMEM((FPT,), jnp.int32),
            pltpu.SMEM((1,), jnp.int32),
            *[pltpu.SemaphoreType.DMA for _ in range(NSLOT)],
            *[pltpu.SemaphoreType.DMA for _ in range(NSLOT)],
        ],
    )
    def k(values_hbm, ids_hbm, firsts_hbm, out_hbm, acc,
          vb0, vb1, vb2, vb3, ib0, ib1, ib2, ib3, zbuf, cbuf, cnt_smem,
          ls0, ls1, ls2, ls3, ss0, ss1, ss2, ss3):
        c = lax.axis_index("c")
        s = lax.axis_index("s")
        vbufs = (vb0, vb1, vb2, vb3)
        ibufs = (ib0, ib1, ib2, ib3)
        lsems = (ls0, ls1, ls2, ls3)
        ssems = (ss0, ss1, ss2, ss3)

        @pl.loop(0, 64)
        def _zero(i):
            for j in range(D // 16):
                zbuf[i, pl.ds(j * 16, 16)] = jnp.zeros((16,), jnp.float32)

        off = 0
        for sz in WSIZES:
            pltpu.async_copy(
                zbuf.at[pl.ds(0, sz)], acc.at[pl.ds(s * TILE_W + off, sz)],
                ss0)
            off += sz

        @pl.when(s == NS - 1)
        def _zero_rem():
            pltpu.async_copy(
                zbuf.at[pl.ds(0, WREM)], acc.at[pl.ds(WREM_OFF, WREM)], ss0)

        @pl.when(s == 0)
        def _zero_cnt():
            cnt_smem[0] = 0

        plsc.subcore_barrier()
        pltpu.sync_copy(firsts_hbm.at[pl.ds(s * FPT, FPT)], cbuf)
        ib0[pl.ds(0, 16)] = jnp.zeros((16,), jnp.int32)

        @pl.loop(0, FPT // 16)
        def _cnt(i):
            iv = cbuf[pl.ds(i * 16, 16)]
            one = jnp.ones((16,), jnp.int32)
            zero = jnp.zeros((16,), jnp.int32)
            ib0[pl.ds(0, 16)] = (
                ib0[pl.ds(0, 16)] + jnp.where(iv < HALF, one, zero))

        cv = ib0[pl.ds(0, 16)]
        my_cnt = cv[0]
        for l in range(1, 16):
            my_cnt = my_cnt + cv[l]
        plsc.fetch_and_add(cnt_smem, my_cnt, subcore_id=0)
        plsc.subcore_barrier()
        c0 = plsc.fetch_and_add(cnt_smem, 0, subcore_id=0)

        cs = jnp.where(c == 0, 0, jnp.maximum(c0 - 1, 0))
        ce = jnp.where(c == 0, c0, NCHUNKS)
        nk = jnp.maximum(ce - cs - s, 0)
        nk = (nk + NS - 1) // NS
        lo = c * HALF

        def _load(kk, slot):
            @pl.when(kk < nk)
            def _():
                base = (cs + s + NS * kk) * CHUNK
                pltpu.async_copy(
                    values_hbm.at[pl.ds(base, CHUNK)], vbufs[slot],
                    lsems[slot])
                pltpu.async_copy(
                    ids_hbm.at[pl.ds(base, CHUNK)], ibufs[slot], lsems[slot])

        for kk in range(LOOKAHEAD):
            _load(kk, kk)

        off = 0
        for sz in WSIZES:
            pltpu.make_async_copy(
                zbuf.at[pl.ds(0, sz)], acc.at[pl.ds(s * TILE_W + off, sz)],
                ss0).wait()
            off += sz

        @pl.when(s == NS - 1)
        def _zero_rem_wait():
            pltpu.make_async_copy(
                zbuf.at[pl.ds(0, WREM)], acc.at[pl.ds(WREM_OFF, WREM)],
                ss0).wait()

        plsc.subcore_barrier()

        nk4 = ((nk + NSLOT - 1) // NSLOT) * NSLOT

        @pl.loop(0, nk4, step=NSLOT)
        def _body(kk0):
            for b in range(NSLOT):
                vb, ib = vbufs[b], ibufs[b]
                kk = kk0 + b
                nxt = (b + LOOKAHEAD) % NSLOT

                @pl.when(kk < nk)
                def _():
                    pltpu.make_async_copy(
                        values_hbm.at[pl.ds(0, CHUNK)], vb, lsems[b]).wait()
                    pltpu.make_async_copy(
                        ids_hbm.at[pl.ds(0, CHUNK)], ib, lsems[b]).wait()
                    for j in range(CHUNK // 16):
                        iv = ib[pl.ds(j * 16, 16)]
                        loc = iv - lo
                        ok = (loc >= 0) & (loc < HALF)
                        trash = jnp.full((16,), TRASH, jnp.int32)
                        ib[pl.ds(j * 16, 16)] = jnp.where(ok, loc, trash)
                    pltpu.async_copy(vb, acc.at[ib], ssems[b], add=True)

                    @pl.when(kk >= 1)
                    def _():
                        pltpu.make_async_copy(
                            vbufs[nxt], acc.at[ibufs[nxt]], ssems[nxt]).wait()

                    _load(kk + LOOKAHEAD, nxt)

        @pl.when(nk > 0)
        def _drain():
            last = (nk - 1) % NSLOT
            for b in range(NSLOT):
                @pl.when(last == b)
                def _():
                    pltpu.make_async_copy(
                        vbufs[b], acc.at[ibufs[b]], ssems[b]).wait()

        plsc.subcore_barrier()

        pltpu.sync_copy(
            acc.at[pl.ds(s * TILE_W, TILE_W)],
            out_hbm.at[pl.ds(lo + s * TILE_W, TILE_W)])

        @pl.when(s == NS - 1)
        def _wb_rem():
            pltpu.sync_copy(
                acc.at[pl.ds(WREM_OFF, WREM)],
                out_hbm.at[pl.ds(lo + WREM_OFF, WREM)])

    return k(values, ids, firsts)


def kernel(values, segment_ids):
    ids = segment_ids.astype(jnp.int32)
    firsts = jnp.pad(
        ids.reshape(NCHUNKS, CHUNK)[:, 0], (0, FPAD - NCHUNKS),
        constant_values=NSEG)
    return _sc_segment_sum(values, ids, firsts)
